# baseline (speedup 1.0000x reference)
import functools

import jax
import jax.numpy as jnp
from jax import lax
from jax.experimental import pallas as pl
from jax.experimental.pallas import tpu as pltpu
from jax.experimental.pallas import tpu_sc as plsc

L = 16


def _sc_body(
    tok_hbm, k_hbm, out_hbm,
    tok0_v, tok1_v, out0_v, out1_v, carr_v, k_v,
    sem_in0, sem_in1, sem_out0, sem_out1,
):
    B, S = tok_hbm.shape
    n_chunks = S // L
    cid = lax.axis_index("c")
    sid = lax.axis_index("s")
    wid = sid * 2 + cid
    row0 = wid * 2
    row1 = row0 + 1

    in0 = pltpu.async_copy(tok_hbm.at[row0], tok0_v, sem_in0)
    in1 = pltpu.async_copy(tok_hbm.at[row1], tok1_v, sem_in1)
    pltpu.sync_copy(k_hbm, k_v)
    k_vec = k_v[...]
    lane = lax.iota(jnp.int32, L)
    zero_v = jnp.zeros((L,), jnp.int32)
    one_v = jnp.ones((L,), jnp.int32)

    def process_row(tok_v, out_v):
        c0 = tok_v[pl.ds(0, L)]
        tok_v[pl.ds(0, L)] = jnp.where(lane == 0, 0, c0)
        cl = tok_v[pl.ds(S - L, L)]
        tok_v[pl.ds(S - L, L)] = jnp.where(lane == L - 1, 0, cl)

        @plsc.parallel_loop(0, n_chunks, carry=zero_v, unroll=32)
        def acc_v(i, acc):
            v = tok_v[pl.ds(i * L, L)]
            mi = jnp.minimum(v, one_v)
            w = plsc.cumsum(mi)
            carr_v[i] = w[L - 1]
            return acc + mi

        n_s = jnp.sum(acc_v)

        @plsc.parallel_loop(0, n_chunks, carry=(jnp.int32(0), lane + n_s), unroll=32)
        def _(i, carry):
            c, np_v = carry
            v = tok_v[pl.ds(i * L, L)]
            mi = jnp.minimum(v, one_v)
            cn = plsc.cumsum(mi) + c
            rank = jnp.where(mi > 0, cn, np_v - cn)
            out_v[pl.ds(i * L, L)] = jnp.where(rank < k_vec, one_v, zero_v)
            return c + carr_v[i], np_v + L

        o0 = out_v[pl.ds(0, L)]
        out_v[pl.ds(0, L)] = jnp.where(lane == 0, one_v, o0)

    in0.wait()
    process_row(tok0_v, out0_v)
    o0 = pltpu.async_copy(out0_v, out_hbm.at[row0], sem_out0)
    in1.wait()
    process_row(tok1_v, out1_v)
    o1 = pltpu.async_copy(out1_v, out_hbm.at[row1], sem_out1)
    o0.wait()
    o1.wait()


def kernel(token_sequence, embedding_sequence, compression_rate):
    del embedding_sequence
    B, S = token_sequence.shape
    k = jnp.maximum(jnp.asarray(S * (1 - compression_rate)), 1).astype(jnp.int32)
    k_arr = jnp.full((L,), k, jnp.int32)
    mesh = plsc.VectorSubcoreMesh(core_axis_name="c", subcore_axis_name="s")
    sc_call = functools.partial(
        pl.kernel,
        out_type=jax.ShapeDtypeStruct((B, S), jnp.int32),
        mesh=mesh,
        scratch_types=[
            pltpu.VMEM((S,), jnp.int32),
            pltpu.VMEM((S,), jnp.int32),
            pltpu.VMEM((S,), jnp.int32),
            pltpu.VMEM((S,), jnp.int32),
            pltpu.SMEM((S // L,), jnp.int32),
            pltpu.VMEM((L,), jnp.int32),
            pltpu.SemaphoreType.DMA,
            pltpu.SemaphoreType.DMA,
            pltpu.SemaphoreType.DMA,
            pltpu.SemaphoreType.DMA,
        ],
        compiler_params=pltpu.CompilerParams(needs_layout_passes=False),
    )
    out_i32 = sc_call(_sc_body)(token_sequence, k_arr)
    y_hard = out_i32.astype(jnp.bool_)
    return (y_hard, y_hard)


# speedup vs baseline: 1.0204x; 1.0204x over previous
import functools

import jax
import jax.numpy as jnp
from jax import lax
from jax.experimental import pallas as pl
from jax.experimental.pallas import tpu as pltpu
from jax.experimental.pallas import tpu_sc as plsc

L = 16


def _sc_body(
    tok_hbm, k_hbm, out_hbm,
    tok0_v, tok1_v, out0_v, out1_v, carr_v, k_v,
    sem_in0, sem_in1, sem_out0, sem_out1,
):
    B, S = tok_hbm.shape
    n_chunks = S // L
    cid = lax.axis_index("c")
    sid = lax.axis_index("s")
    wid = sid * 2 + cid
    row0 = wid * 2
    row1 = row0 + 1

    in0 = pltpu.async_copy(tok_hbm.at[row0], tok0_v, sem_in0)
    in1 = pltpu.async_copy(tok_hbm.at[row1], tok1_v, sem_in1)
    pltpu.sync_copy(k_hbm, k_v)
    k_vec = k_v[...]
    lane = lax.iota(jnp.int32, L)
    zero_v = jnp.zeros((L,), jnp.int32)
    one_v = jnp.ones((L,), jnp.int32)

    def process_row(tok_v, out_v):
        c0 = tok_v[pl.ds(0, L)]
        tok_v[pl.ds(0, L)] = jnp.where(lane == 0, 0, c0)
        cl = tok_v[pl.ds(S - L, L)]
        tok_v[pl.ds(S - L, L)] = jnp.where(lane == L - 1, 0, cl)

        @plsc.parallel_loop(0, n_chunks, carry=zero_v, unroll=8)
        def acc_v(i, acc):
            v = tok_v[pl.ds(i * L, L)]
            mi = jnp.minimum(v, one_v)
            w = plsc.cumsum(mi)
            carr_v[i] = w[L - 1]
            return acc + mi

        n_s = jnp.sum(acc_v)

        @plsc.parallel_loop(0, n_chunks, carry=(jnp.int32(0), lane + n_s), unroll=8)
        def _(i, carry):
            c, np_v = carry
            v = tok_v[pl.ds(i * L, L)]
            mi = jnp.minimum(v, one_v)
            cn = plsc.cumsum(mi) + c
            rank = jnp.where(mi > 0, cn, np_v - cn)
            out_v[pl.ds(i * L, L)] = jnp.where(rank < k_vec, one_v, zero_v)
            return c + carr_v[i], np_v + L

        o0 = out_v[pl.ds(0, L)]
        out_v[pl.ds(0, L)] = jnp.where(lane == 0, one_v, o0)

    in0.wait()
    process_row(tok0_v, out0_v)
    o0 = pltpu.async_copy(out0_v, out_hbm.at[row0], sem_out0)
    in1.wait()
    process_row(tok1_v, out1_v)
    o1 = pltpu.async_copy(out1_v, out_hbm.at[row1], sem_out1)
    o0.wait()
    o1.wait()


def kernel(token_sequence, embedding_sequence, compression_rate):
    del embedding_sequence
    B, S = token_sequence.shape
    k = jnp.maximum(jnp.asarray(S * (1 - compression_rate)), 1).astype(jnp.int32)
    k_arr = jnp.full((L,), k, jnp.int32)
    mesh = plsc.VectorSubcoreMesh(core_axis_name="c", subcore_axis_name="s")
    sc_call = functools.partial(
        pl.kernel,
        out_type=jax.ShapeDtypeStruct((B, S), jnp.int32),
        mesh=mesh,
        scratch_types=[
            pltpu.VMEM((S,), jnp.int32),
            pltpu.VMEM((S,), jnp.int32),
            pltpu.VMEM((S,), jnp.int32),
            pltpu.VMEM((S,), jnp.int32),
            pltpu.SMEM((S // L,), jnp.int32),
            pltpu.VMEM((L,), jnp.int32),
            pltpu.SemaphoreType.DMA,
            pltpu.SemaphoreType.DMA,
            pltpu.SemaphoreType.DMA,
            pltpu.SemaphoreType.DMA,
        ],
        compiler_params=pltpu.CompilerParams(needs_layout_passes=False),
    )
    out_i32 = sc_call(_sc_body)(token_sequence, k_arr)
    y_hard = out_i32.astype(jnp.bool_)
    return (y_hard, y_hard)
